# baseline (speedup 1.0000x reference)
import random

import jax
import jax.numpy as jnp
import numpy as np
from jax.experimental import pallas as pl
from jax.experimental.pallas import tpu as pltpu

_ADJ_LISTS = [[1, 2, 3, 4], [0, 2, 3], [0, 1, 4], [0, 1], [0, 2]]
_NUM_LAYERS = 2


def _unique_neighs(nodes, rng, num_sample=2):
    to_neighs = [_ADJ_LISTS[int(n)] for n in nodes]
    samp = [set(rng.sample(tn, num_sample)) if len(tn) >= num_sample else set(tn)
            for tn in to_neighs]
    samp = [sn | {nodes[i]} for i, sn in enumerate(samp)]
    unique_list = list(set.union(*samp))
    unique_dict = {n: i for i, n in enumerate(unique_list)}
    return samp, unique_dict, unique_list


def _structure(n_nodes):
    rng = random.Random(0)
    nodes_batch = list(range(n_nodes))
    lower = list(nodes_batch)
    layers = [(lower,)]
    for _ in range(_NUM_LAYERS):
        samp, udict, ulist = _unique_neighs(lower, rng, num_sample=2)
        layers.insert(0, (ulist, samp, udict))
        lower = ulist

    mixers = []
    nb_idx = nodes_batch
    for index in range(1, _NUM_LAYERS + 1):
        nb = layers[index][0]
        unique_list, samp_neighs, unique_dict = layers[index - 1]
        sn = [samp_neighs[i] - {nb[i]} for i in range(len(samp_neighs))]
        n_pre = n_nodes if index == 1 else len(layers[index - 1][0])
        if n_pre == len(unique_dict):
            emb_sel = np.eye(n_pre, dtype=np.float32)
        else:
            emb_sel = np.zeros((len(unique_list), n_pre), dtype=np.float32)
            emb_sel[np.arange(len(unique_list)), unique_list] = 1.0
        mask = np.zeros((len(sn), len(unique_dict)), dtype=np.float32)
        for i, s in enumerate(sn):
            for node in s:
                mask[i, unique_dict[node]] = 1.0
        mask = mask / mask.sum(1, keepdims=True)
        S = mask @ emb_sel
        if index > 1:
            nb_idx = [unique_dict[xn] for xn in nb]
        else:
            nb_idx = nb
        A = np.zeros((len(nb_idx), n_pre), dtype=np.float32)
        A[np.arange(len(nb_idx)), nb_idx] = 1.0
        mixers.append((A, S))
    return mixers, nb_idx


def _const2d(vals_np):
    r, c = vals_np.shape
    row = jax.lax.broadcasted_iota(jnp.int32, (r, c), 0)
    col = jax.lax.broadcasted_iota(jnp.int32, (r, c), 1)
    acc = jnp.zeros((r, c), jnp.float32)
    for i in range(r):
        for j in range(c):
            v = float(vals_np[i, j])
            if v != 0.0:
                acc = acc + jnp.where((row == i) & (col == j), v, 0.0)
    return acc


def _combine(pre, A, S):
    if np.array_equal(A, np.eye(A.shape[0], A.shape[1])):
        self_feats = pre
    else:
        self_feats = jnp.dot(_const2d(A), pre,
                             preferred_element_type=jnp.float32)
    agg = jnp.dot(_const2d(S), pre, preferred_element_type=jnp.float32)
    return jnp.concatenate([self_feats, agg], axis=1)


def _chunk_out(comb, w_chunk):
    return jax.nn.relu(jax.lax.dot_general(
        comb, w_chunk, (((1,), (1,)), ((), ())),
        preferred_element_type=jnp.float32))


def _make_fwd_kernel(mixers, n, out_size):
    def _fwd_kernel(x_h, w1_h, w2_h, fc_h, fcb_h, flat_ref, lp_ref,
                    xv, w1v, w2v, fcv, fcbv,
                    sx, s1, s2, sfc, sfcb):
        nsplit = 2
        chunk = w1v.shape[0] // nsplit
        c_x = pltpu.make_async_copy(x_h, xv, sx)
        c_x.start()
        c1 = [pltpu.make_async_copy(w1_h.at[pl.ds(i * chunk, chunk)],
                                    w1v.at[pl.ds(i * chunk, chunk)],
                                    s1.at[i])
              for i in range(nsplit)]
        c2 = [pltpu.make_async_copy(w2_h.at[pl.ds(i * chunk, chunk)],
                                    w2v.at[pl.ds(i * chunk, chunk)],
                                    s2.at[i])
              for i in range(nsplit)]
        for c in c1:
            c.start()
        cfc = pltpu.make_async_copy(fc_h, fcv, sfc)
        cfc.start()
        cfcb = pltpu.make_async_copy(fcb_h, fcbv, sfcb)
        cfcb.start()

        c_x.wait()
        comb = _combine(xv[...], *mixers[0])
        for c in c1:
            c.wait()
        for c in c2:
            c.start()
        pre = _chunk_out(comb, w1v[...])
        comb = _combine(pre, *mixers[1])
        for c in c2:
            c.wait()
        pre = _chunk_out(comb, w2v[...])
        flat = jnp.concatenate([pre[i:i + 1, :] for i in range(n)], axis=1)
        flat_ref[...] = flat
        cfc.wait()
        cfcb.wait()
        l0 = jnp.sum(flat * fcv[0:1, :]) + fcbv[0, 0]
        l1 = jnp.sum(flat * fcv[1:2, :]) + fcbv[0, 1]
        m = jnp.maximum(l0, l1)
        lse = m + jnp.log(jnp.exp(l0 - m) + jnp.exp(l1 - m))
        lane = jax.lax.broadcasted_iota(jnp.int32, (1, 2), 1)
        lp_ref[...] = jnp.where(lane == 0, l0, l1) - lse
    return _fwd_kernel


def kernel(input, adj, W1, W2, fc_w, fc_b):
    n = int(adj.shape[0])
    in_size = input.shape[1]
    out_size = W1.shape[0]
    mixers, nb_idx = _structure(n)

    fcb = fc_b.reshape(1, 2)

    pre_flat, lp = pl.pallas_call(
        _make_fwd_kernel(mixers, n, out_size),
        in_specs=[pl.BlockSpec(memory_space=pl.ANY)] * 5,
        out_shape=(
            jax.ShapeDtypeStruct((1, n * out_size), jnp.float32),
            jax.ShapeDtypeStruct((1, 2), jnp.float32),
        ),
        scratch_shapes=[
            pltpu.VMEM((n, in_size), jnp.float32),
            pltpu.VMEM(W1.shape, jnp.float32),
            pltpu.VMEM(W2.shape, jnp.float32),
            pltpu.VMEM(fc_w.shape, jnp.float32),
            pltpu.VMEM((1, 2), jnp.float32),
        ] + [pltpu.SemaphoreType.DMA,
             pltpu.SemaphoreType.DMA((2,)),
             pltpu.SemaphoreType.DMA((2,)),
             pltpu.SemaphoreType.DMA,
             pltpu.SemaphoreType.DMA],
    )(input, W1, W2, fc_w, fcb)

    return jnp.asarray(nb_idx, dtype=jnp.int32), pre_flat, lp


# speedup vs baseline: 1.2090x; 1.2090x over previous
import random

import jax
import jax.numpy as jnp
import numpy as np
from jax.experimental import pallas as pl
from jax.experimental.pallas import tpu as pltpu

_ADJ_LISTS = [[1, 2, 3, 4], [0, 2, 3], [0, 1, 4], [0, 1], [0, 2]]
_NUM_LAYERS = 2


def _unique_neighs(nodes, rng, num_sample=2):
    to_neighs = [_ADJ_LISTS[int(n)] for n in nodes]
    samp = [set(rng.sample(tn, num_sample)) if len(tn) >= num_sample else set(tn)
            for tn in to_neighs]
    samp = [sn | {nodes[i]} for i, sn in enumerate(samp)]
    unique_list = list(set.union(*samp))
    unique_dict = {n: i for i, n in enumerate(unique_list)}
    return samp, unique_dict, unique_list


def _structure(n_nodes):
    rng = random.Random(0)
    nodes_batch = list(range(n_nodes))
    lower = list(nodes_batch)
    layers = [(lower,)]
    for _ in range(_NUM_LAYERS):
        samp, udict, ulist = _unique_neighs(lower, rng, num_sample=2)
        layers.insert(0, (ulist, samp, udict))
        lower = ulist

    mixers = []
    nb_idx = nodes_batch
    for index in range(1, _NUM_LAYERS + 1):
        nb = layers[index][0]
        unique_list, samp_neighs, unique_dict = layers[index - 1]
        sn = [samp_neighs[i] - {nb[i]} for i in range(len(samp_neighs))]
        n_pre = n_nodes if index == 1 else len(layers[index - 1][0])
        if n_pre == len(unique_dict):
            emb_sel = np.eye(n_pre, dtype=np.float32)
        else:
            emb_sel = np.zeros((len(unique_list), n_pre), dtype=np.float32)
            emb_sel[np.arange(len(unique_list)), unique_list] = 1.0
        mask = np.zeros((len(sn), len(unique_dict)), dtype=np.float32)
        for i, s in enumerate(sn):
            for node in s:
                mask[i, unique_dict[node]] = 1.0
        mask = mask / mask.sum(1, keepdims=True)
        S = mask @ emb_sel
        if index > 1:
            nb_idx = [unique_dict[xn] for xn in nb]
        else:
            nb_idx = nb
        A = np.zeros((len(nb_idx), n_pre), dtype=np.float32)
        A[np.arange(len(nb_idx)), nb_idx] = 1.0
        mixers.append((A, S))
    return mixers, nb_idx


def _const2d(vals_np):
    r, c = vals_np.shape
    row = jax.lax.broadcasted_iota(jnp.int32, (r, c), 0)
    col = jax.lax.broadcasted_iota(jnp.int32, (r, c), 1)
    acc = jnp.zeros((r, c), jnp.float32)
    for i in range(r):
        for j in range(c):
            v = float(vals_np[i, j])
            if v != 0.0:
                acc = acc + jnp.where((row == i) & (col == j), v, 0.0)
    return acc


def _combine(pre, A, S):
    if np.array_equal(A, np.eye(A.shape[0], A.shape[1])):
        self_feats = pre
    else:
        self_feats = jnp.dot(_const2d(A), pre,
                             preferred_element_type=jnp.float32)
    agg = jnp.dot(_const2d(S), pre, preferred_element_type=jnp.float32)
    return jnp.concatenate([self_feats, agg], axis=1)


def _chunk_out(comb, w_chunk):
    return jax.nn.relu(jax.lax.dot_general(
        comb, w_chunk, (((1,), (1,)), ((), ())),
        preferred_element_type=jnp.float32))


def _make_fwd_kernel(mixers, n, out_size):
    def _fwd_kernel(x_h, w1_h, w2_h, fc_h, fcb_h, flat_ref, lp_ref,
                    xv, w1v, w2v, fcv, fcbv,
                    sx, s1, s2, sfc, sfcb):
        c_x = pltpu.make_async_copy(x_h, xv, sx)
        c_x.start()
        c1 = pltpu.make_async_copy(w1_h, w1v, s1)
        c1.start()
        c2 = pltpu.make_async_copy(w2_h, w2v, s2)
        c2.start()
        cfc = pltpu.make_async_copy(fc_h, fcv, sfc)
        cfc.start()
        cfcb = pltpu.make_async_copy(fcb_h, fcbv, sfcb)
        cfcb.start()

        c_x.wait()
        comb = _combine(xv[...], *mixers[0])
        c1.wait()
        pre = _chunk_out(comb, w1v[...])
        comb = _combine(pre, *mixers[1])
        c2.wait()
        pre = _chunk_out(comb, w2v[...])
        flat = jnp.concatenate([pre[i:i + 1, :] for i in range(n)], axis=1)
        flat_ref[...] = flat
        cfc.wait()
        cfcb.wait()
        l0 = jnp.sum(flat * fcv[0:1, :]) + fcbv[0, 0]
        l1 = jnp.sum(flat * fcv[1:2, :]) + fcbv[0, 1]
        m = jnp.maximum(l0, l1)
        lse = m + jnp.log(jnp.exp(l0 - m) + jnp.exp(l1 - m))
        lane = jax.lax.broadcasted_iota(jnp.int32, (1, 2), 1)
        lp_ref[...] = jnp.where(lane == 0, l0, l1) - lse
    return _fwd_kernel


def kernel(input, adj, W1, W2, fc_w, fc_b):
    n = int(adj.shape[0])
    in_size = input.shape[1]
    out_size = W1.shape[0]
    mixers, nb_idx = _structure(n)

    fcb = fc_b.reshape(1, 2)

    pre_flat, lp = pl.pallas_call(
        _make_fwd_kernel(mixers, n, out_size),
        in_specs=[pl.BlockSpec(memory_space=pl.ANY)] * 5,
        out_shape=(
            jax.ShapeDtypeStruct((1, n * out_size), jnp.float32),
            jax.ShapeDtypeStruct((1, 2), jnp.float32),
        ),
        scratch_shapes=[
            pltpu.VMEM((n, in_size), jnp.float32),
            pltpu.VMEM(W1.shape, jnp.float32),
            pltpu.VMEM(W2.shape, jnp.float32),
            pltpu.VMEM(fc_w.shape, jnp.float32),
            pltpu.VMEM((1, 2), jnp.float32),
        ] + [pltpu.SemaphoreType.DMA] * 5,
    )(input, W1, W2, fc_w, fcb)

    return jnp.asarray(nb_idx, dtype=jnp.int32), pre_flat, lp


# speedup vs baseline: 1.2254x; 1.0135x over previous
import random

import jax
import jax.numpy as jnp
import numpy as np
from jax.experimental import pallas as pl
from jax.experimental.pallas import tpu as pltpu

_ADJ_LISTS = [[1, 2, 3, 4], [0, 2, 3], [0, 1, 4], [0, 1], [0, 2]]
_NUM_LAYERS = 2


def _unique_neighs(nodes, rng, num_sample=2):
    to_neighs = [_ADJ_LISTS[int(n)] for n in nodes]
    samp = [set(rng.sample(tn, num_sample)) if len(tn) >= num_sample else set(tn)
            for tn in to_neighs]
    samp = [sn | {nodes[i]} for i, sn in enumerate(samp)]
    unique_list = list(set.union(*samp))
    unique_dict = {n: i for i, n in enumerate(unique_list)}
    return samp, unique_dict, unique_list


def _structure(n_nodes):
    rng = random.Random(0)
    nodes_batch = list(range(n_nodes))
    lower = list(nodes_batch)
    layers = [(lower,)]
    for _ in range(_NUM_LAYERS):
        samp, udict, ulist = _unique_neighs(lower, rng, num_sample=2)
        layers.insert(0, (ulist, samp, udict))
        lower = ulist

    mixers = []
    nb_idx = nodes_batch
    for index in range(1, _NUM_LAYERS + 1):
        nb = layers[index][0]
        unique_list, samp_neighs, unique_dict = layers[index - 1]
        sn = [samp_neighs[i] - {nb[i]} for i in range(len(samp_neighs))]
        n_pre = n_nodes if index == 1 else len(layers[index - 1][0])
        if n_pre == len(unique_dict):
            emb_sel = np.eye(n_pre, dtype=np.float32)
        else:
            emb_sel = np.zeros((len(unique_list), n_pre), dtype=np.float32)
            emb_sel[np.arange(len(unique_list)), unique_list] = 1.0
        mask = np.zeros((len(sn), len(unique_dict)), dtype=np.float32)
        for i, s in enumerate(sn):
            for node in s:
                mask[i, unique_dict[node]] = 1.0
        mask = mask / mask.sum(1, keepdims=True)
        S = mask @ emb_sel
        if index > 1:
            nb_idx = [unique_dict[xn] for xn in nb]
        else:
            nb_idx = nb
        A = np.zeros((len(nb_idx), n_pre), dtype=np.float32)
        A[np.arange(len(nb_idx)), nb_idx] = 1.0
        mixers.append((A, S))
    return mixers, nb_idx


def _const2d(vals_np):
    r, c = vals_np.shape
    row = jax.lax.broadcasted_iota(jnp.int32, (r, c), 0)
    col = jax.lax.broadcasted_iota(jnp.int32, (r, c), 1)
    acc = jnp.zeros((r, c), jnp.float32)
    for i in range(r):
        for j in range(c):
            v = float(vals_np[i, j])
            if v != 0.0:
                acc = acc + jnp.where((row == i) & (col == j), v, 0.0)
    return acc


def _mix_rows(pre, M):
    rows = []
    for i in range(M.shape[0]):
        acc = None
        for j in range(M.shape[1]):
            v = float(M[i, j])
            if v == 0.0:
                continue
            term = pre[j:j + 1, :] if v == 1.0 else v * pre[j:j + 1, :]
            acc = term if acc is None else acc + term
        if acc is None:
            acc = jnp.zeros_like(pre[0:1, :])
        rows.append(acc)
    return jnp.concatenate(rows, axis=0)


def _combine(pre, A, S):
    if np.array_equal(A, np.eye(A.shape[0], A.shape[1])):
        self_feats = pre
    else:
        self_feats = _mix_rows(pre, A)
    agg = _mix_rows(pre, S)
    return jnp.concatenate([self_feats, agg], axis=1)


def _chunk_out(comb, w_chunk):
    return jax.nn.relu(jax.lax.dot_general(
        comb, w_chunk, (((1,), (1,)), ((), ())),
        preferred_element_type=jnp.float32))


def _make_fwd_kernel(mixers, n, out_size):
    def _fwd_kernel(x_h, w1_h, w2_h, fc_h, fcb_h, flat_ref, lp_ref,
                    xv, w1v, w2v, fcv, fcbv,
                    sx, s1, s2, sfc, sfcb):
        c_x = pltpu.make_async_copy(x_h, xv, sx)
        c_x.start()
        c1 = pltpu.make_async_copy(w1_h, w1v, s1)
        c1.start()
        c2 = pltpu.make_async_copy(w2_h, w2v, s2)
        c2.start()
        cfc = pltpu.make_async_copy(fc_h, fcv, sfc)
        cfc.start()
        cfcb = pltpu.make_async_copy(fcb_h, fcbv, sfcb)
        cfcb.start()

        c_x.wait()
        comb = _combine(xv[...], *mixers[0])
        c1.wait()
        pre = _chunk_out(comb, w1v[...])
        comb = _combine(pre, *mixers[1])
        c2.wait()
        pre = _chunk_out(comb, w2v[...])
        cfc.wait()
        cfcb.wait()
        sz = pre.shape[1]
        p0 = None
        p1 = None
        for i in range(n):
            row = pre[i:i + 1, :]
            flat_ref[:, i * sz:(i + 1) * sz] = row
            t0 = row * fcv[0:1, i * sz:(i + 1) * sz]
            t1 = row * fcv[1:2, i * sz:(i + 1) * sz]
            p0 = t0 if p0 is None else p0 + t0
            p1 = t1 if p1 is None else p1 + t1
        l0 = jnp.sum(p0) + fcbv[0, 0]
        l1 = jnp.sum(p1) + fcbv[0, 1]
        m = jnp.maximum(l0, l1)
        lse = m + jnp.log(jnp.exp(l0 - m) + jnp.exp(l1 - m))
        lane = jax.lax.broadcasted_iota(jnp.int32, (1, 2), 1)
        lp_ref[...] = jnp.where(lane == 0, l0, l1) - lse
    return _fwd_kernel


def kernel(input, adj, W1, W2, fc_w, fc_b):
    n = int(adj.shape[0])
    in_size = input.shape[1]
    out_size = W1.shape[0]
    mixers, nb_idx = _structure(n)

    fcb = fc_b.reshape(1, 2)

    pre_flat, lp = pl.pallas_call(
        _make_fwd_kernel(mixers, n, out_size),
        in_specs=[pl.BlockSpec(memory_space=pl.ANY)] * 5,
        out_shape=(
            jax.ShapeDtypeStruct((1, n * out_size), jnp.float32),
            jax.ShapeDtypeStruct((1, 2), jnp.float32),
        ),
        scratch_shapes=[
            pltpu.VMEM((n, in_size), jnp.float32),
            pltpu.VMEM(W1.shape, jnp.float32),
            pltpu.VMEM(W2.shape, jnp.float32),
            pltpu.VMEM(fc_w.shape, jnp.float32),
            pltpu.VMEM((1, 2), jnp.float32),
        ] + [pltpu.SemaphoreType.DMA] * 5,
    )(input, W1, W2, fc_w, fcb)

    return jnp.asarray(nb_idx, dtype=jnp.int32), pre_flat, lp
